# baseline (speedup 1.0000x reference)
import functools

import jax
import jax.numpy as jnp
from jax import lax
from jax.experimental import pallas as pl
from jax.experimental.pallas import tpu as pltpu
from jax.experimental.pallas import tpu_sc as plsc

_N = 10000
_E = 320000
_DIN = 128
_DH = 64
_DP = 128
_NC = 2
_NS = 16
_NW = _NC * _NS
_EPW = _E // _NW
_CHUNK = 80
_NCHUNK = _EPW // _CHUNK
_SLAB = 624
_TAIL_BASE = _NS * _SLAB
_TAIL = _N - _TAIL_BASE
_CH = 128
_BR = 16
_NBLK = 5
_EPW_PAD = _NBLK * _BR * _CH
_NPAD = _EPW_PAD - _EPW
_NP = _N + 16

@functools.cache
def _mesh():
    return plsc.VectorSubcoreMesh(
        core_axis_name="c", subcore_axis_name="s", num_cores=_NC, num_subcores=_NS
    )


def _deg_body(dst_hbm, out_hbm, dacc, dst_all, ones_v, zb):
    cid = lax.axis_index("c")
    sid = lax.axis_index("s")
    wid = sid * _NC + cid
    pltpu.sync_copy(dst_hbm.at[wid], dst_all)

    def _z(i, _):
        zb[pl.ds(i * 16, 16)] = jnp.zeros((16,), jnp.float32)
        return 0

    lax.fori_loop(0, _SLAB // 16, _z, 0)

    def _o(i, _):
        ones_v[pl.ds(i * 16, 16)] = jnp.ones((16,), jnp.float32)
        return 0

    lax.fori_loop(0, _CHUNK // 16, _o, 0)

    base = sid * _SLAB
    pltpu.sync_copy(zb, dacc.at[pl.ds(base, _SLAB)])

    @pl.when(sid == _NS - 1)
    def _():
        pltpu.sync_copy(zb.at[pl.ds(0, _TAIL)], dacc.at[pl.ds(_TAIL_BASE, _TAIL)])

    plsc.subcore_barrier()

    def _step(c, _):
        pltpu.sync_copy(ones_v, dacc.at[dst_all.at[c]], add=True)
        return 0

    lax.fori_loop(0, _NCHUNK, _step, 0)
    plsc.subcore_barrier()

    pltpu.sync_copy(dacc.at[pl.ds(base, _SLAB)], zb)
    pltpu.sync_copy(zb, out_hbm.at[pl.ds(cid * _N + base, _SLAB)])

    @pl.when(sid == _NS - 1)
    def _():
        pltpu.sync_copy(dacc.at[pl.ds(_TAIL_BASE, _TAIL)], zb.at[pl.ds(0, _TAIL)])
        pltpu.sync_copy(zb.at[pl.ds(0, _TAIL)],
                        out_hbm.at[pl.ds(cid * _N + _TAIL_BASE, _TAIL)])


@functools.cache
def _deg():
    return pl.kernel(
        _deg_body,
        out_type=jax.ShapeDtypeStruct((_NC * _N,), jnp.float32),
        mesh=_mesh(),
        scratch_types=[
            pltpu.VMEM_SHARED((_N,), jnp.float32),
            pltpu.VMEM((_NCHUNK, _CHUNK), jnp.int32),
            pltpu.VMEM((_CHUNK,), jnp.float32),
            pltpu.VMEM((_SLAB,), jnp.float32),
        ],
    )


def _agg_body(y_hbm, src_hbm, dst_hbm, out_hbm, acc, src_blk, dst_blk,
              rows_a, rows_b, sem_a, sem_b, sem_sa, sem_sb):
    cid = lax.axis_index("c")
    sid = lax.axis_index("s")
    wid = sid * _NC + cid

    def _z(r, _):
        for j in range(_DP // 16):
            rows_a[r, pl.ds(j * 16, 16)] = jnp.zeros((16,), jnp.float32)
        return 0

    lax.fori_loop(0, _CH, _z, 0)

    base = sid * _SLAB
    _REM = _SLAB % _CH
    for k in range(_SLAB // _CH):
        pltpu.sync_copy(rows_a, acc.at[pl.ds(base + k * _CH, _CH)])
    pltpu.sync_copy(rows_a.at[pl.ds(0, _REM)],
                    acc.at[pl.ds(base + _SLAB - _REM, _REM)])

    @pl.when(sid == _NS - 1)
    def _():
        pltpu.sync_copy(rows_a.at[pl.ds(0, _TAIL)], acc.at[pl.ds(_TAIL_BASE, _TAIL)])

    plsc.subcore_barrier()

    def _issue_g(r, buf, sem):
        pltpu.async_copy(y_hbm.at[src_blk.at[r]], buf, sem)

    def _wait_g(buf, sem):
        pltpu.make_async_copy(y_hbm.at[src_blk.at[0]], buf, sem).wait()

    def _issue_s(r, buf, sem):
        pltpu.async_copy(buf, acc.at[dst_blk.at[r]], sem, add=True)

    def _wait_s(buf, sem):
        pltpu.make_async_copy(buf, acc.at[dst_blk.at[0]], sem).wait()

    def _blk(b, _):
        pltpu.sync_copy(src_hbm.at[wid, b], src_blk)
        pltpu.sync_copy(dst_hbm.at[wid, b], dst_blk)
        _issue_g(0, rows_a, sem_a)
        _issue_g(1, rows_b, sem_b)

        def _step(i, _):
            _wait_g(rows_a, sem_a)
            _issue_s(2 * i, rows_a, sem_sa)
            _wait_g(rows_b, sem_b)
            _issue_s(2 * i + 1, rows_b, sem_sb)
            _wait_s(rows_a, sem_sa)

            @pl.when(i < _BR // 2 - 1)
            def _():
                _issue_g(2 * i + 2, rows_a, sem_a)

            _wait_s(rows_b, sem_sb)

            @pl.when(i < _BR // 2 - 1)
            def _():
                _issue_g(2 * i + 3, rows_b, sem_b)

            return 0

        lax.fori_loop(0, _BR // 2, _step, 0)
        return 0

    lax.fori_loop(0, _NBLK, _blk, 0)
    plsc.subcore_barrier()

    for k in range(_SLAB // _CH):
        off = base + k * _CH
        pltpu.sync_copy(acc.at[pl.ds(off, _CH)], rows_a)
        pltpu.sync_copy(rows_a, out_hbm.at[cid, pl.ds(off, _CH)])
    off = base + _SLAB - _REM
    pltpu.sync_copy(acc.at[pl.ds(off, _REM)], rows_a.at[pl.ds(0, _REM)])
    pltpu.sync_copy(rows_a.at[pl.ds(0, _REM)], out_hbm.at[cid, pl.ds(off, _REM)])

    @pl.when(sid == _NS - 1)
    def _():
        pltpu.sync_copy(acc.at[pl.ds(_TAIL_BASE, _TAIL)], rows_b.at[pl.ds(0, _TAIL)])
        pltpu.sync_copy(rows_b.at[pl.ds(0, _TAIL)],
                        out_hbm.at[cid, pl.ds(_TAIL_BASE, _TAIL)])


@functools.cache
def _agg():
    return pl.kernel(
        _agg_body,
        out_type=jax.ShapeDtypeStruct((_NC, _N, _DP), jnp.float32),
        mesh=_mesh(),
        scratch_types=[
            pltpu.VMEM_SHARED((_NP, _DP), jnp.float32),
            pltpu.VMEM((_BR, _CH), jnp.int32),
            pltpu.VMEM((_BR, _CH), jnp.int32),
            pltpu.VMEM((_CH, _DP), jnp.float32),
            pltpu.VMEM((_CH, _DP), jnp.float32),
            pltpu.SemaphoreType.DMA,
            pltpu.SemaphoreType.DMA,
            pltpu.SemaphoreType.DMA,
            pltpu.SemaphoreType.DMA,
        ],
    )


def _tc1_body(degp_ref, x_ref, w_ref, y_ref, dis_ref):
    deg = degp_ref[:, 0:1] + degp_ref[:, 1:2] + 1.0
    dis = 1.0 / jnp.sqrt(deg)
    y_ref[pl.ds(0, _N), :] = dis * jnp.dot(x_ref[...], w_ref[...],
                                           preferred_element_type=jnp.float32)
    y_ref[pl.ds(_N, _NP - _N), :] = jnp.zeros((_NP - _N, _DP), jnp.float32)
    dis_ref[...] = dis


_tc1 = pl.pallas_call(
    _tc1_body,
    out_shape=(
        jax.ShapeDtypeStruct((_NP, _DP), jnp.float32),
        jax.ShapeDtypeStruct((_N, 1), jnp.float32),
    ),
)


def _tc_mid_body(p_ref, y_ref, dis_ref, b_ref, w_ref, out_ref):
    s = p_ref[0] + p_ref[1] + y_ref[pl.ds(0, _N), :]
    h = jnp.maximum(dis_ref[...] * s[:, :_DH] + b_ref[...], 0.0)
    out_ref[pl.ds(0, _N), :] = dis_ref[...] * jnp.dot(
        h, w_ref[...], preferred_element_type=jnp.float32)
    out_ref[pl.ds(_N, _NP - _N), :] = jnp.zeros((_NP - _N, _DP), jnp.float32)


_tc_mid = pl.pallas_call(
    _tc_mid_body,
    out_shape=jax.ShapeDtypeStruct((_NP, _DP), jnp.float32),
)


def _tc_final_body(p_ref, y_ref, dis_ref, b_ref, wq_ref, bq_ref, q_ref):
    s = p_ref[0] + p_ref[1] + y_ref[pl.ds(0, _N), :]
    h = jnp.maximum(dis_ref[...] * s[:, :_DH] + b_ref[...], 0.0)
    q_ref[...] = jnp.dot(h, wq_ref[...],
                         preferred_element_type=jnp.float32) + bq_ref[...]


def _make_tc_final(a):
    return pl.pallas_call(
        _tc_final_body,
        out_shape=jax.ShapeDtypeStruct((_N, a), jnp.float32),
    )


def kernel(x, edge_index, W1, b1, W2, b2, W3, b3, Wq, bq):
    src3 = edge_index[0].reshape(_NW, _NCHUNK, _CHUNK)
    dst3 = edge_index[1].reshape(_NW, _NCHUNK, _CHUNK)

    padi = _N + (jnp.arange(_NPAD, dtype=jnp.int32) % (_NP - _N))
    padw = jnp.broadcast_to(padi, (_NW, _NPAD))
    src4 = jnp.concatenate([edge_index[0].reshape(_NW, _EPW), padw], axis=1)
    src4 = src4.reshape(_NW, _NBLK, _BR, _CH)
    dst4 = jnp.concatenate([edge_index[1].reshape(_NW, _EPW), padw], axis=1)
    dst4 = dst4.reshape(_NW, _NBLK, _BR, _CH)

    degp = _deg()(dst3)
    degp_t = jnp.transpose(degp.reshape(_NC, _N))

    pad = ((0, 0), (0, _DP - _DH))
    W1p, W2p, W3p = (jnp.pad(W, pad) for W in (W1, W2, W3))

    y1, dis = _tc1(degp_t, x, W1p)
    p = _agg()(y1, src4, dst4)
    y2 = _tc_mid(p, y1, dis, b1.reshape(1, _DH), W2p)
    p = _agg()(y2, src4, dst4)
    y3 = _tc_mid(p, y2, dis, b2.reshape(1, _DH), W3p)
    p = _agg()(y3, src4, dst4)
    q = _make_tc_final(Wq.shape[1])(p, y3, dis, b3.reshape(1, _DH), Wq,
                                    bq.reshape(1, Wq.shape[1]))
    return q


# speedup vs baseline: 1.5451x; 1.5451x over previous
import functools

import jax
import jax.experimental.layout as jexl
import jax.numpy as jnp
from jax import lax
from jax.experimental import pallas as pl
from jax.experimental.pallas import tpu as pltpu
from jax.experimental.pallas import tpu_sc as plsc

_N = 10000
_E = 320000
_DIN = 128
_DH = 64
_DP = 64
_NC = 2
_NS = 16
_NW = _NC * _NS
_EPW = _E // _NW
_CHUNK = 80
_NCHUNK = _EPW // _CHUNK
_SLAB = 624
_TAIL_BASE = _NS * _SLAB
_TAIL = _N - _TAIL_BASE
_CH = 128
_BR = 16
_NBLK = 5
_EPW_PAD = _NBLK * _BR * _CH
_NPAD = _EPW_PAD - _EPW
_NP = _N + 16

@functools.cache
def _mesh():
    return plsc.VectorSubcoreMesh(
        core_axis_name="c", subcore_axis_name="s", num_cores=_NC, num_subcores=_NS
    )


def _deg_body(dst_hbm, out_hbm, dacc, dst_all, ones_v, zb):
    cid = lax.axis_index("c")
    sid = lax.axis_index("s")
    wid = sid * _NC + cid
    pltpu.sync_copy(dst_hbm.at[wid], dst_all)

    def _z(i, _):
        zb[pl.ds(i * 16, 16)] = jnp.zeros((16,), jnp.float32)
        return 0

    lax.fori_loop(0, _SLAB // 16, _z, 0)

    def _o(i, _):
        ones_v[pl.ds(i * 16, 16)] = jnp.ones((16,), jnp.float32)
        return 0

    lax.fori_loop(0, _CHUNK // 16, _o, 0)

    base = sid * _SLAB
    pltpu.sync_copy(zb, dacc.at[pl.ds(base, _SLAB)])

    @pl.when(sid == _NS - 1)
    def _():
        pltpu.sync_copy(zb.at[pl.ds(0, _TAIL)], dacc.at[pl.ds(_TAIL_BASE, _TAIL)])

    plsc.subcore_barrier()

    def _step(c, _):
        pltpu.sync_copy(ones_v, dacc.at[dst_all.at[c]], add=True)
        return 0

    lax.fori_loop(0, _NCHUNK, _step, 0)
    plsc.subcore_barrier()

    pltpu.sync_copy(dacc.at[pl.ds(base, _SLAB)], zb)
    pltpu.sync_copy(zb, out_hbm.at[pl.ds(cid * _N + base, _SLAB)])

    @pl.when(sid == _NS - 1)
    def _():
        pltpu.sync_copy(dacc.at[pl.ds(_TAIL_BASE, _TAIL)], zb.at[pl.ds(0, _TAIL)])
        pltpu.sync_copy(zb.at[pl.ds(0, _TAIL)],
                        out_hbm.at[pl.ds(cid * _N + _TAIL_BASE, _TAIL)])


@functools.cache
def _deg():
    return pl.kernel(
        _deg_body,
        out_type=jax.ShapeDtypeStruct((_NC * _N,), jnp.float32),
        mesh=_mesh(),
        scratch_types=[
            pltpu.VMEM_SHARED((_N,), jnp.float32),
            pltpu.VMEM((_NCHUNK, _CHUNK), jnp.int32),
            pltpu.VMEM((_CHUNK,), jnp.float32),
            pltpu.VMEM((_SLAB,), jnp.float32),
        ],
    )


def _agg_body(y_hbm, src_hbm, dst_hbm, out_hbm, acc, src_blk, dst_blk,
              rows_a, rows_b, sem_a, sem_b, sem_sa, sem_sb):
    cid = lax.axis_index("c")
    sid = lax.axis_index("s")
    wid = sid * _NC + cid

    def _z(r, _):
        for j in range(_DP // 16):
            rows_a[r, pl.ds(j * 16, 16)] = jnp.zeros((16,), jnp.float32)
        return 0

    lax.fori_loop(0, _CH, _z, 0)

    base = sid * _SLAB
    _REM = _SLAB % _CH
    for k in range(_SLAB // _CH):
        pltpu.sync_copy(rows_a, acc.at[pl.ds(base + k * _CH, _CH)])
    pltpu.sync_copy(rows_a.at[pl.ds(0, _REM)],
                    acc.at[pl.ds(base + _SLAB - _REM, _REM)])

    @pl.when(sid == _NS - 1)
    def _():
        pltpu.sync_copy(rows_a.at[pl.ds(0, _TAIL)], acc.at[pl.ds(_TAIL_BASE, _TAIL)])

    plsc.subcore_barrier()

    def _issue_g(r, buf, sem):
        pltpu.async_copy(y_hbm.at[src_blk.at[r]], buf, sem)

    def _wait_g(buf, sem):
        pltpu.make_async_copy(y_hbm.at[src_blk.at[0]], buf, sem).wait()

    def _issue_s(r, buf, sem):
        pltpu.async_copy(buf, acc.at[dst_blk.at[r]], sem, add=True)

    def _wait_s(buf, sem):
        pltpu.make_async_copy(buf, acc.at[dst_blk.at[0]], sem).wait()

    def _blk(b, _):
        pltpu.sync_copy(src_hbm.at[wid, b], src_blk)
        pltpu.sync_copy(dst_hbm.at[wid, b], dst_blk)
        _issue_g(0, rows_a, sem_a)
        _issue_g(1, rows_b, sem_b)

        def _step(i, _):
            _wait_g(rows_a, sem_a)
            pltpu.sync_copy(rows_a, acc.at[dst_blk.at[2 * i]], add=True)

            @pl.when(i < _BR // 2 - 1)
            def _():
                _issue_g(2 * i + 2, rows_a, sem_a)

            _wait_g(rows_b, sem_b)
            pltpu.sync_copy(rows_b, acc.at[dst_blk.at[2 * i + 1]], add=True)

            @pl.when(i < _BR // 2 - 1)
            def _():
                _issue_g(2 * i + 3, rows_b, sem_b)

            return 0

        lax.fori_loop(0, _BR // 2, _step, 0)
        return 0

    lax.fori_loop(0, _NBLK, _blk, 0)
    plsc.subcore_barrier()

    for k in range(_SLAB // _CH):
        off = base + k * _CH
        pltpu.sync_copy(acc.at[pl.ds(off, _CH)], rows_a)
        pltpu.sync_copy(rows_a, out_hbm.at[cid, pl.ds(off, _CH)])
    off = base + _SLAB - _REM
    pltpu.sync_copy(acc.at[pl.ds(off, _REM)], rows_a.at[pl.ds(0, _REM)])
    pltpu.sync_copy(rows_a.at[pl.ds(0, _REM)], out_hbm.at[cid, pl.ds(off, _REM)])

    @pl.when(sid == _NS - 1)
    def _():
        pltpu.sync_copy(acc.at[pl.ds(_TAIL_BASE, _TAIL)], rows_b.at[pl.ds(0, _TAIL)])
        pltpu.sync_copy(rows_b.at[pl.ds(0, _TAIL)],
                        out_hbm.at[cid, pl.ds(_TAIL_BASE, _TAIL)])


@functools.cache
def _agg():
    return pl.kernel(
        _agg_body,
        out_type=jax.ShapeDtypeStruct((_NC, _N, _DP), jnp.float32),
        mesh=_mesh(),
        scratch_types=[
            pltpu.VMEM_SHARED((_NP, _DP), jnp.float32),
            pltpu.VMEM((_BR, _CH), jnp.int32),
            pltpu.VMEM((_BR, _CH), jnp.int32),
            pltpu.VMEM((_CH, _DP), jnp.float32),
            pltpu.VMEM((_CH, _DP), jnp.float32),
            pltpu.SemaphoreType.DMA,
            pltpu.SemaphoreType.DMA,
            pltpu.SemaphoreType.DMA,
            pltpu.SemaphoreType.DMA,
        ],
    )


def _tc1_body(degp_ref, x_ref, w_ref, y_ref, dis_ref):
    deg = degp_ref[:, 0:1] + degp_ref[:, 1:2] + 1.0
    dis = 1.0 / jnp.sqrt(deg)
    y_ref[pl.ds(0, _N), :] = dis * jnp.dot(x_ref[...], w_ref[...],
                                           preferred_element_type=jnp.float32)
    y_ref[pl.ds(_N, _NP - _N), :] = jnp.zeros((_NP - _N, _DP), jnp.float32)
    dis_ref[...] = dis


_tc1 = pl.pallas_call(
    _tc1_body,
    out_shape=(
        jax.ShapeDtypeStruct((_NP, _DP), jnp.float32),
        jax.ShapeDtypeStruct((_N, 1), jnp.float32),
    ),
)


def _tc_mid_body(p_ref, y_ref, dis_ref, b_ref, w_ref, out_ref):
    s = p_ref[0] + p_ref[1] + y_ref[pl.ds(0, _N), :]
    h = jnp.maximum(dis_ref[...] * s[:, :_DH] + b_ref[...], 0.0)
    out_ref[pl.ds(0, _N), :] = dis_ref[...] * jnp.dot(
        h, w_ref[...], preferred_element_type=jnp.float32)
    out_ref[pl.ds(_N, _NP - _N), :] = jnp.zeros((_NP - _N, _DP), jnp.float32)


_tc_mid = pl.pallas_call(
    _tc_mid_body,
    out_shape=jax.ShapeDtypeStruct((_NP, _DP), jnp.float32),
)


def _tc_final_body(p_ref, y_ref, dis_ref, b_ref, wq_ref, bq_ref, q_ref):
    s = p_ref[0] + p_ref[1] + y_ref[pl.ds(0, _N), :]
    h = jnp.maximum(dis_ref[...] * s[:, :_DH] + b_ref[...], 0.0)
    q_ref[...] = jnp.dot(h, wq_ref[...],
                         preferred_element_type=jnp.float32) + bq_ref[...]


def _make_tc_final(a):
    return pl.pallas_call(
        _tc_final_body,
        out_shape=jax.ShapeDtypeStruct((_N, a), jnp.float32),
    )


def kernel(x, edge_index, W1, b1, W2, b2, W3, b3, Wq, bq):
    src3 = edge_index[0].reshape(_NW, _NCHUNK, _CHUNK)
    dst3 = edge_index[1].reshape(_NW, _NCHUNK, _CHUNK)

    padi = _N + (jnp.arange(_NPAD, dtype=jnp.int32) % (_NP - _N))
    padw = jnp.broadcast_to(padi, (_NW, _NPAD))
    src4 = jnp.concatenate([edge_index[0].reshape(_NW, _EPW), padw], axis=1)
    src4 = src4.reshape(_NW, _NBLK, _BR, _CH)
    dst4 = jnp.concatenate([edge_index[1].reshape(_NW, _EPW), padw], axis=1)
    dst4 = dst4.reshape(_NW, _NBLK, _BR, _CH)

    degp = _deg()(dst3)
    degp_t = jnp.transpose(degp.reshape(_NC, _N))

    pad = ((0, 0), (0, _DP - _DH))
    W1p, W2p, W3p = (jnp.pad(W, pad) for W in (W1, W2, W3))

    _sc_layout = jexl.Layout((0, 1), tiling=((16,),))

    y1, dis = _tc1(degp_t, x, W1p)
    p = _agg()(jexl.with_layout_constraint(y1, _sc_layout), src4, dst4)
    y2 = _tc_mid(p, y1, dis, b1.reshape(1, _DH), W2p)
    p = _agg()(jexl.with_layout_constraint(y2, _sc_layout), src4, dst4)
    y3 = _tc_mid(p, y2, dis, b2.reshape(1, _DH), W3p)
    p = _agg()(jexl.with_layout_constraint(y3, _sc_layout), src4, dst4)
    q = _make_tc_final(Wq.shape[1])(p, y3, dis, b3.reshape(1, _DH), Wq,
                                    bq.reshape(1, Wq.shape[1]))
    return q
